# baseline (speedup 1.0000x reference)
import jax
import jax.numpy as jnp
from jax import lax
from jax.experimental import pallas as pl
from jax.experimental.pallas import tpu as pltpu

_BETA_THRESHOLD = 0.3
_DIST_THRESHOLD = 0.5
_N = 20000
_D0 = 20
_D1 = 8
_D2 = 128
_NPAD = _D0 * _D1 * _D2

_BIG_IDX = 2**30
_SHAPE = (_D0, _D1, _D2)


def _condense_kernel(spk_ref, m_ref, cx_ref, cy_ref, cz_ref, pk_ref):
    flat = (lax.broadcasted_iota(jnp.int32, _SHAPE, 0) * (_D1 * _D2)
            + lax.broadcasted_iota(jnp.int32, _SHAPE, 1) * _D2
            + lax.broadcasted_iota(jnp.int32, _SHAPE, 2))

    m_count = m_ref[0]
    cx = cx_ref[:]
    cy = cy_ref[:]
    cz = cz_ref[:]

    pk_ref[:] = jnp.full(_SHAPE, -1, jnp.int32)

    def body(state):
        k, p = state
        base = p * jnp.int32(4)
        ax = spk_ref[base]
        ay = spk_ref[base + 1]
        az = spk_ref[base + 2]
        ra = spk_ref[base + 3] * jnp.float32(_DIST_THRESHOLD)

        dx = cx - ax
        dy = cy - ay
        dz = cz - az
        d = jnp.sqrt(dx * dx + dy * dy + dz * dz)
        inrad = d <= ra
        pk = pk_ref[:]
        unas = pk < 0
        newpk = k * jnp.int32(32768) + p
        pk_ref[:] = jnp.where(unas & inrad, newpk, pk)
        cand = jnp.where(unas & jnp.logical_not(inrad), flat,
                         jnp.int32(_BIG_IDX))
        p2 = jnp.min(jnp.min(cand, axis=0))
        return k + jnp.int32(1), p2

    lax.while_loop(lambda s: s[1] < m_count, body,
                   (jnp.int32(0), jnp.int32(0)))


def kernel(pred_ccoords, pred_beta, pred_dist):
    beta = pred_beta.reshape(-1)
    order = jnp.argsort(-beta, stable=True).astype(jnp.int32)
    orig_f = lax.bitcast_convert_type(jnp.arange(_N, dtype=jnp.int32),
                                      jnp.float32)
    packed = jnp.stack(
        [pred_ccoords[:, 0], pred_ccoords[:, 1], pred_ccoords[:, 2],
         pred_dist.reshape(-1), orig_f], axis=1)
    packed_s = packed[order]

    pad = _NPAD - _N
    cx = jnp.pad(packed_s[:, 0], (0, pad), constant_values=1e30)
    cy = jnp.pad(packed_s[:, 1], (0, pad), constant_values=1e30)
    cz = jnp.pad(packed_s[:, 2], (0, pad), constant_values=1e30)
    sdist = jnp.pad(packed_s[:, 3], (0, pad), constant_values=0.0)
    smem_pack = jnp.stack([cx, cy, cz, sdist], axis=1).reshape(-1)

    m_arr = jnp.sum(beta > jnp.float32(_BETA_THRESHOLD)).astype(
        jnp.int32).reshape(1)

    smem_spec = pl.BlockSpec(memory_space=pltpu.SMEM)
    pk3d = pl.pallas_call(
        _condense_kernel,
        out_shape=jax.ShapeDtypeStruct(_SHAPE, jnp.int32),
        in_specs=[smem_spec, smem_spec] + [pl.BlockSpec()] * 3,
    )(smem_pack, m_arr,
      cx.reshape(_SHAPE), cy.reshape(_SHAPE), cz.reshape(_SHAPE))

    pk_s = pk3d.reshape(-1)[:_N]
    pko = jnp.zeros((_N,), jnp.int32).at[order].set(pk_s)
    valid = pko >= 0
    q = pko // jnp.int32(32768)
    assign = jnp.where(valid, q, -1)
    alpha_spos = jnp.where(valid, pko - q * jnp.int32(32768), 0)
    comb = jnp.take(packed_s, alpha_spos, axis=0)
    alpha_idx = jnp.where(
        valid, lax.bitcast_convert_type(comb[:, 4], jnp.int32), -1)
    cond_coords = jnp.where(valid[:, None], comb[:, :3],
                            jnp.zeros((_N, 3), jnp.float32))
    return assign, alpha_idx, cond_coords


# speedup vs baseline: 1.0007x; 1.0007x over previous
import jax
import jax.numpy as jnp
from jax import lax
from jax.experimental import pallas as pl
from jax.experimental.pallas import tpu as pltpu

_BETA_THRESHOLD = 0.3
_DIST_THRESHOLD = 0.5
_N = 20000
_D0 = 20
_D1 = 8
_D2 = 128
_NPAD = _D0 * _D1 * _D2

_BIG_IDX = 2**30
_SHAPE = (_D0, _D1, _D2)


def _condense_kernel(spk_ref, m_ref, cx_ref, cy_ref, cz_ref, pk_ref):
    flat = (lax.broadcasted_iota(jnp.int32, _SHAPE, 0) * (_D1 * _D2)
            + lax.broadcasted_iota(jnp.int32, _SHAPE, 1) * _D2
            + lax.broadcasted_iota(jnp.int32, _SHAPE, 2))

    m_count = m_ref[0]
    cx = cx_ref[:]
    cy = cy_ref[:]
    cz = cz_ref[:]

    pk_ref[:] = jnp.full(_SHAPE, -1, jnp.int32)

    def body(state):
        k, p = state
        base = p * jnp.int32(4)
        ax = spk_ref[base]
        ay = spk_ref[base + 1]
        az = spk_ref[base + 2]
        ra = spk_ref[base + 3] * jnp.float32(_DIST_THRESHOLD)

        dx = cx - ax
        dy = cy - ay
        dz = cz - az
        d = jnp.sqrt(dx * dx + dy * dy + dz * dz)
        inrad = d <= ra
        pk = pk_ref[:]
        unas = pk < 0
        newpk = k * jnp.int32(32768) + p
        pk_ref[:] = jnp.where(unas & inrad, newpk, pk)
        cand = jnp.where(unas & jnp.logical_not(inrad), flat,
                         jnp.int32(_BIG_IDX))
        p2 = jnp.min(jnp.min(cand, axis=0))
        return k + jnp.int32(1), p2

    lax.while_loop(lambda s: s[1] < m_count, body,
                   (jnp.int32(0), jnp.int32(0)))


def kernel(pred_ccoords, pred_beta, pred_dist):
    beta = pred_beta.reshape(-1)
    order = jnp.argsort(-beta, stable=True).astype(jnp.int32)
    orig_f = lax.bitcast_convert_type(
        jnp.arange(_N, dtype=jnp.int32) + jnp.int32(0x40000000), jnp.float32)
    packed = jnp.stack(
        [pred_ccoords[:, 0], pred_ccoords[:, 1], pred_ccoords[:, 2],
         pred_dist.reshape(-1), orig_f], axis=1)
    packed_s = packed[order]

    pad = _NPAD - _N
    cx = jnp.pad(packed_s[:, 0], (0, pad), constant_values=1e30)
    cy = jnp.pad(packed_s[:, 1], (0, pad), constant_values=1e30)
    cz = jnp.pad(packed_s[:, 2], (0, pad), constant_values=1e30)
    sdist = jnp.pad(packed_s[:, 3], (0, pad), constant_values=0.0)
    smem_pack = jnp.stack([cx, cy, cz, sdist], axis=1).reshape(-1)

    m_arr = jnp.sum(beta > jnp.float32(_BETA_THRESHOLD)).astype(
        jnp.int32).reshape(1)

    smem_spec = pl.BlockSpec(memory_space=pltpu.SMEM)
    pk3d = pl.pallas_call(
        _condense_kernel,
        out_shape=jax.ShapeDtypeStruct(_SHAPE, jnp.int32),
        in_specs=[smem_spec, smem_spec] + [pl.BlockSpec()] * 3,
    )(smem_pack, m_arr,
      cx.reshape(_SHAPE), cy.reshape(_SHAPE), cz.reshape(_SHAPE))

    pk_s = pk3d.reshape(-1)[:_N]
    pko = jnp.zeros((_N,), jnp.int32).at[order].set(pk_s)
    valid = pko >= 0
    q = pko // jnp.int32(32768)
    assign = jnp.where(valid, q, -1)
    alpha_spos = jnp.where(valid, pko - q * jnp.int32(32768), 0)
    comb = jnp.take(packed_s, alpha_spos, axis=0)
    alpha_idx = jnp.where(
        valid,
        lax.bitcast_convert_type(comb[:, 4], jnp.int32) - jnp.int32(0x40000000),
        -1)
    cond_coords = jnp.where(valid[:, None], comb[:, :3],
                            jnp.zeros((_N, 3), jnp.float32))
    return assign, alpha_idx, cond_coords


# speedup vs baseline: 1.0679x; 1.0671x over previous
import jax
import jax.numpy as jnp
from jax import lax
from jax.experimental import pallas as pl
from jax.experimental.pallas import tpu as pltpu

_BETA_THRESHOLD = 0.3
_DIST_THRESHOLD = 0.5
_N = 20000
_D0 = 20
_D1 = 8
_D2 = 128
_NPAD = _D0 * _D1 * _D2

_BIG_IDX = 2**30
_SHAPE = (_D0, _D1, _D2)


def _condense_kernel(spk_ref, cx_ref, cy_ref, cz_ref, beta_ref,
                     pk_out, ccx_ref, ccy_ref, ccz_ref, avail_ref):
    flat = (lax.broadcasted_iota(jnp.int32, _SHAPE, 0) * (_D1 * _D2)
            + lax.broadcasted_iota(jnp.int32, _SHAPE, 1) * _D2
            + lax.broadcasted_iota(jnp.int32, _SHAPE, 2))

    cx = cx_ref[:]
    cy = cy_ref[:]
    cz = cz_ref[:]
    beta = beta_ref[:]

    pk_out[:] = jnp.full(_SHAPE, -1, jnp.int32)
    zeros = jnp.zeros(_SHAPE, jnp.float32)
    ccx_ref[:] = zeros
    ccy_ref[:] = zeros
    ccz_ref[:] = zeros
    avail_ref[:] = beta

    def argmax_of(avail):
        m = jnp.max(jnp.max(avail, axis=0))
        a = jnp.min(jnp.min(
            jnp.where(avail == m, flat, jnp.int32(_BIG_IDX)), axis=0))
        return m, a

    m0, a0 = argmax_of(beta)

    def body(state):
        k, a, _m = state
        base = a * jnp.int32(4)
        ax = spk_ref[base]
        ay = spk_ref[base + 1]
        az = spk_ref[base + 2]
        ra = spk_ref[base + 3] * jnp.float32(_DIST_THRESHOLD)

        dx = cx - ax
        dy = cy - ay
        dz = cz - az
        d = jnp.sqrt(dx * dx + dy * dy + dz * dz)
        inrad = d <= ra
        avail = avail_ref[:]
        within = inrad & (avail >= 0.0)
        newpk = k * jnp.int32(32768) + a
        pk_out[:] = jnp.where(within, newpk, pk_out[:])
        ccx_ref[:] = jnp.where(within, ax, ccx_ref[:])
        ccy_ref[:] = jnp.where(within, ay, ccy_ref[:])
        ccz_ref[:] = jnp.where(within, az, ccz_ref[:])
        avail2 = jnp.where(inrad, jnp.float32(-1.0), avail)
        avail_ref[:] = avail2

        m2, a2 = argmax_of(avail2)
        return k + jnp.int32(1), a2, m2

    lax.while_loop(lambda s: s[2] > jnp.float32(_BETA_THRESHOLD), body,
                   (jnp.int32(0), a0, m0))


def kernel(pred_ccoords, pred_beta, pred_dist):
    pad = _NPAD - _N
    cx = jnp.pad(pred_ccoords[:, 0], (0, pad), constant_values=1e30)
    cy = jnp.pad(pred_ccoords[:, 1], (0, pad), constant_values=1e30)
    cz = jnp.pad(pred_ccoords[:, 2], (0, pad), constant_values=1e30)
    beta = jnp.pad(pred_beta.reshape(-1), (0, pad), constant_values=-1.0)
    dist = jnp.pad(pred_dist.reshape(-1), (0, pad), constant_values=0.0)
    smem_pack = jnp.stack([cx, cy, cz, dist], axis=1).reshape(-1)

    smem_spec = pl.BlockSpec(memory_space=pltpu.SMEM)
    out_shape = [
        jax.ShapeDtypeStruct(_SHAPE, jnp.int32),
        jax.ShapeDtypeStruct(_SHAPE, jnp.float32),
        jax.ShapeDtypeStruct(_SHAPE, jnp.float32),
        jax.ShapeDtypeStruct(_SHAPE, jnp.float32),
    ]
    pk3d, ccx, ccy, ccz = pl.pallas_call(
        _condense_kernel,
        out_shape=out_shape,
        in_specs=[smem_spec] + [pl.BlockSpec()] * 4,
        scratch_shapes=[pltpu.VMEM(_SHAPE, jnp.float32)],
    )(smem_pack, cx.reshape(_SHAPE), cy.reshape(_SHAPE), cz.reshape(_SHAPE),
      beta.reshape(_SHAPE))

    pk = pk3d.reshape(-1)[:_N]
    valid = pk >= 0
    q = pk // jnp.int32(32768)
    assign = jnp.where(valid, q, -1)
    alpha_idx = jnp.where(valid, pk - q * jnp.int32(32768), -1)
    cond_coords = jnp.stack(
        [ccx.reshape(-1)[:_N], ccy.reshape(-1)[:_N], ccz.reshape(-1)[:_N]],
        axis=-1)
    return assign, alpha_idx, cond_coords
